# baseline (speedup 1.0000x reference)
import functools

import jax
import jax.numpy as jnp
from jax import lax
from jax.experimental import pallas as pl
from jax.experimental.pallas import tpu as pltpu
from jax.experimental.pallas import tpu_sc as plsc

NUM_VEC = 8192
DIM = 64
NC = 2
NS = 16
NW = NC * NS
ROWS_PER_W = NUM_VEC // NW


@functools.partial(
    pl.kernel,
    mesh=plsc.VectorSubcoreMesh(core_axis_name="c", subcore_axis_name="s"),
    out_type=jax.ShapeDtypeStruct((NUM_VEC, DIM), jnp.float32),
)
def _sc_copy(emb_hbm, out_hbm):
    wid = lax.axis_index("s") * NC + lax.axis_index("c")
    base = wid * ROWS_PER_W
    pltpu.sync_copy(
        emb_hbm.at[pl.ds(base, ROWS_PER_W)],
        out_hbm.at[pl.ds(base, ROWS_PER_W)],
    )


def kernel(embeddings):
    return _sc_copy(embeddings)


# speedup vs baseline: 5.4557x; 5.4557x over previous
import functools

import jax
import jax.numpy as jnp
from jax import lax
from jax.experimental import pallas as pl
from jax.experimental.pallas import tpu as pltpu
from jax.experimental.pallas import tpu_sc as plsc

NUM_VEC = 8192
DIM = 64
NC = 2
NS = 16
NW = NC * NS
ROWS_PER_W = NUM_VEC // NW


@functools.partial(
    pl.kernel,
    mesh=plsc.VectorSubcoreMesh(core_axis_name="c", subcore_axis_name="s"),
    out_type=jax.ShapeDtypeStruct((NUM_VEC, DIM), jnp.float32),
    scratch_types=[pltpu.VMEM((ROWS_PER_W, DIM), jnp.float32)],
)
def _sc_copy(emb_hbm, out_hbm, buf_v):
    wid = lax.axis_index("s") * NC + lax.axis_index("c")
    base = wid * ROWS_PER_W
    pltpu.sync_copy(emb_hbm.at[pl.ds(base, ROWS_PER_W)], buf_v)
    pltpu.sync_copy(buf_v, out_hbm.at[pl.ds(base, ROWS_PER_W)])


def kernel(embeddings):
    return _sc_copy(embeddings)
